# baseline (speedup 1.0000x reference)
import functools

import jax
import jax.numpy as jnp
from jax import lax
from jax.experimental import pallas as pl
from jax.experimental.pallas import tpu as pltpu
from jax.experimental.pallas import tpu_sc as plsc

_BQ = 8
_BM = 64


def _sc_gather(table, idx):
    V, D = table.shape
    B = idx.shape[0]
    info = plsc.get_sparse_core_info()
    nw = info.num_cores * info.num_subcores
    b_per_w = B // nw
    mesh = plsc.VectorSubcoreMesh(core_axis_name="c", subcore_axis_name="s")

    @functools.partial(
        pl.kernel,
        mesh=mesh,
        out_type=jax.ShapeDtypeStruct((B, D), jnp.float32),
        scratch_types=[
            pltpu.VMEM((b_per_w,), jnp.int32),
            pltpu.VMEM((b_per_w, D), jnp.float32),
            pltpu.SemaphoreType.DMA,
        ],
    )
    def gather_kernel(table_hbm, idx_hbm, out_hbm, idx_v, rows_v, sem):
        wid = lax.axis_index("s") * info.num_cores + lax.axis_index("c")
        base = wid * b_per_w
        pltpu.sync_copy(idx_hbm.at[pl.ds(base, b_per_w)], idx_v)
        pltpu.async_copy(table_hbm.at[idx_v], rows_v, sem).wait()
        pltpu.sync_copy(rows_v, out_hbm.at[pl.ds(base, b_per_w)])

    return gather_kernel(table, idx)


_BR = 32


def _height_body(lat_ref, g_ref, out_ref, gt_ref):
    for j in range(_BR):
        gt_ref[:, j * 32:(j + 1) * 32] = g_ref[j]
    out_ref[...] = lax.dot_general(
        lat_ref[...],
        gt_ref[...],
        (((1,), (0,)), ((), ())),
        preferred_element_type=jnp.float32,
    )


def _baseline_body(lat_ref, wbt_ref, out_ref):
    out_ref[...] = lax.dot_general(
        lat_ref[...],
        wbt_ref[...],
        (((1,), (0,)), ((), ())),
        preferred_element_type=jnp.float32,
    )


def kernel(latent, regions_oi, delta_height_weight, delta_baseline_weight):
    n_cells, n_latent = latent.shape
    n_regions, _, n_comp = delta_height_weight.shape
    n_oi = regions_oi.shape[0]

    table = delta_height_weight.reshape(n_regions, n_latent * n_comp)
    gathered = _sc_gather(table, regions_oi).reshape(n_oi, n_latent, n_comp)
    n_flat = n_oi * n_comp
    h2 = pl.pallas_call(
        _height_body,
        grid=(n_oi // _BR,),
        in_specs=[
            pl.BlockSpec((n_cells, n_latent), lambda r: (0, 0)),
            pl.BlockSpec((_BR, n_latent, n_comp), lambda r: (r, 0, 0)),
        ],
        out_specs=pl.BlockSpec((n_cells, _BR * n_comp), lambda r: (0, r)),
        out_shape=jax.ShapeDtypeStruct((n_cells, n_flat), jnp.float32),
        scratch_shapes=[pltpu.VMEM((n_latent, _BR * n_comp), jnp.float32)],
    )(latent, gathered)
    delta_height = h2.reshape(n_cells, n_oi, n_comp)

    n_full = delta_baseline_weight.shape[0]
    wbt = delta_baseline_weight.T
    delta_baseline = pl.pallas_call(
        _baseline_body,
        grid=(n_cells // _BM,),
        in_specs=[
            pl.BlockSpec((_BM, n_latent), lambda m: (m, 0)),
            pl.BlockSpec((n_latent, n_full), lambda m: (0, 0)),
        ],
        out_specs=pl.BlockSpec((_BM, n_full), lambda m: (m, 0)),
        out_shape=jax.ShapeDtypeStruct((n_cells, n_full), jnp.float32),
    )(latent, wbt)

    return (delta_height, delta_baseline)
